# Initial kernel scaffold; baseline (speedup 1.0000x reference)
import jax
import jax.numpy as jnp
from jax.experimental import pallas as pl


def kernel(x, W_q, gamma, beta, keys, values):
    raise NotImplementedError("write your pallas kernel here")


# speedup vs baseline: 4.8621x; 4.8621x over previous
import dataclasses
import functools
import math

import jax
import jax.numpy as jnp
from jax import lax
from jax.experimental import pallas as pl
from jax.experimental.pallas import tpu as pltpu
from jax.experimental.pallas import tpu_sc as plsc

DIM = 1024
HEADS = 4
NUM_KEYS = 256
TOPK = 32
HALF = 128
DIM_QUERY = 1024
T = 2048
TB = 256
NCAND = 128

_CAND_COUNTS = [min(TOPK, TOPK // (i + 1)) for i in range(TOPK)]


def _qstats_body(x_ref, wqt_ref, q_ref, stats_ref):
    i = pl.program_id(0)
    q = jax.lax.dot_general(
        x_ref[...], wqt_ref[...], (((1,), (0,)), ((), ())),
        preferred_element_type=jnp.float32)
    q_ref[...] = q
    s = jnp.sum(q, axis=0, keepdims=True)
    s2 = jnp.sum(q * q, axis=0, keepdims=True)

    @pl.when(i == 0)
    def _():
        stats_ref[0:1, :] = s
        stats_ref[1:2, :] = s2

    @pl.when(i > 0)
    def _():
        stats_ref[0:1, :] += s
        stats_ref[1:2, :] += s2


def _compute_q_stats(xs, wqt):
    return pl.pallas_call(
        _qstats_body,
        grid=(T // TB,),
        in_specs=[
            pl.BlockSpec((TB, DIM), lambda i: (i, 0)),
            pl.BlockSpec((DIM, DIM_QUERY), lambda i: (0, 0)),
        ],
        out_specs=[
            pl.BlockSpec((TB, DIM_QUERY), lambda i: (i, 0)),
            pl.BlockSpec((2, DIM_QUERY), lambda i: (0, 0)),
        ],
        out_shape=[
            jax.ShapeDtypeStruct((T, DIM_QUERY), jnp.float32),
            jax.ShapeDtypeStruct((2, DIM_QUERY), jnp.float32),
        ],
    )(xs, wqt)


def _topk_extract_ref(A_ref, k, vals_ref, cols_ref=None,
                      V_ref=None, vsel_ref=None):
    R, W = A_ref.shape
    iota = lax.broadcasted_iota(jnp.int32, (R, W), 1)
    kiota = lax.broadcasted_iota(jnp.int32, (R, k), 1)

    def body(i, _):
        A = A_ref[...]
        m = jnp.max(A, axis=1, keepdims=True)
        c = jnp.min(jnp.where(A == m, iota, W), axis=1, keepdims=True)
        vals_ref[...] = jnp.where(kiota == i, m, vals_ref[...])
        if cols_ref is not None:
            cols_ref[...] = jnp.where(kiota == i, c, cols_ref[...])
        if V_ref is not None:
            vv = jnp.sum(jnp.where(iota == c, V_ref[...], 0.0),
                         axis=1, keepdims=True)
            vsel_ref[...] = jnp.where(kiota == i, vv, vsel_ref[...])
        A_ref[...] = jnp.where(iota == c, -jnp.inf, A)
        return 0

    lax.fori_loop(0, k, body, 0)


def _scores_body(q_ref, stats_ref, gamma_ref, beta_ref, keys_ref,
                 vidx_ref, attn_ref,
                 A1_ref, sv_ref, sc_ref, S4_ref, V4_ref, fv_ref, vs_ref):
    mean = stats_ref[0:1, :] / T
    var = stats_ref[1:2, :] / T - mean * mean
    scale = gamma_ref[...] * lax.rsqrt(var + 1e-5)
    qn = (q_ref[...] - mean) * scale + beta_ref[...]

    for h in range(HEADS):
        for p in range(2):
            qs = qn[:, p * 512 + h * HALF: p * 512 + h * HALF + HALF]
            kt = keys_ref[h, p, :, :]
            g = 2 * h + p
            A1_ref[g * TB:(g + 1) * TB, :] = jax.lax.dot_general(
                qs.astype(jnp.bfloat16), kt, (((1,), (0,)), ((), ())),
                preferred_element_type=jnp.float32)
    _topk_extract_ref(A1_ref, TOPK, sv_ref, cols_ref=sc_ref)

    svals = sv_ref[...]
    scols = sc_ref[...]
    pad = NCAND - sum(_CAND_COUNTS)
    for h in range(HEADS):
        s1 = svals[(2 * h) * TB:(2 * h + 1) * TB, :]
        s2 = svals[(2 * h + 1) * TB:(2 * h + 2) * TB, :]
        i1f = scols[(2 * h) * TB:(2 * h + 1) * TB, :].astype(jnp.float32)
        i2f = scols[(2 * h + 1) * TB:(2 * h + 2) * TB, :].astype(jnp.float32)
        s_pieces = []
        v_pieces = []
        for i, ni in enumerate(_CAND_COUNTS):
            s_pieces.append(s1[:, i:i + 1] + s2[:, :ni])
            v_pieces.append(i1f[:, i:i + 1] * NUM_KEYS + i2f[:, :ni])
        if pad:
            s_pieces.append(jnp.full((TB, pad), -jnp.inf, jnp.float32))
            v_pieces.append(jnp.zeros((TB, pad), jnp.float32))
        S4_ref[h * TB:(h + 1) * TB, :] = jnp.concatenate(s_pieces, axis=1)
        V4_ref[h * TB:(h + 1) * TB, :] = jnp.concatenate(v_pieces, axis=1)

    _topk_extract_ref(S4_ref, TOPK, fv_ref, V_ref=V4_ref, vsel_ref=vs_ref)

    fvals = fv_ref[...]
    e = jnp.exp(fvals - fvals[:, 0:1])
    attn4 = e / jnp.sum(e, axis=1, keepdims=True)
    vsel4 = jnp.clip(vs_ref[...], 0.0, float(NUM_KEYS * NUM_KEYS - 1))
    attn_ref[...] = jnp.concatenate(
        [attn4[h * TB:(h + 1) * TB, :] for h in range(HEADS)], axis=1)
    vidx_ref[...] = jnp.concatenate(
        [vsel4[h * TB:(h + 1) * TB, :] for h in range(HEADS)],
        axis=1).astype(jnp.int32)


def _compute_routing(q, stats, gamma2d, beta2d, keys_t, rows=T):
    nblk = rows // TB
    return pl.pallas_call(
        _scores_body,
        grid=(nblk,),
        in_specs=[
            pl.BlockSpec((TB, DIM_QUERY), lambda i: (i, 0)),
            pl.BlockSpec((2, DIM_QUERY), lambda i: (0, 0)),
            pl.BlockSpec((1, DIM_QUERY), lambda i: (0, 0)),
            pl.BlockSpec((1, DIM_QUERY), lambda i: (0, 0)),
            pl.BlockSpec((HEADS, 2, HALF, NUM_KEYS), lambda i: (0, 0, 0, 0)),
        ],
        out_specs=[
            pl.BlockSpec((TB, HEADS * TOPK), lambda i: (i, 0)),
            pl.BlockSpec((TB, HEADS * TOPK), lambda i: (i, 0)),
        ],
        out_shape=[
            jax.ShapeDtypeStruct((rows, HEADS * TOPK), jnp.int32),
            jax.ShapeDtypeStruct((rows, HEADS * TOPK), jnp.float32),
        ],
        scratch_shapes=[
            pltpu.VMEM((8 * TB, NUM_KEYS), jnp.float32),
            pltpu.VMEM((8 * TB, TOPK), jnp.float32),
            pltpu.VMEM((8 * TB, TOPK), jnp.int32),
            pltpu.VMEM((HEADS * TB, NCAND), jnp.float32),
            pltpu.VMEM((HEADS * TB, NCAND), jnp.float32),
            pltpu.VMEM((HEADS * TB, TOPK), jnp.float32),
            pltpu.VMEM((HEADS * TB, TOPK), jnp.float32),
        ],
    )(q, stats, gamma2d, beta2d, keys_t)


_NC = 2
_NS = 16
_NW = _NC * _NS
_RPW = T // _NW
_K = HEADS * TOPK
_KCH = 32
_NCHUNK = _K // _KCH


def _ebag_sc(values, vidx, attn, rows=T):
    rpw = rows // _NW
    mesh = plsc.VectorSubcoreMesh(core_axis_name="c", subcore_axis_name="s")
    cp = pltpu.CompilerParams()
    if "needs_layout_passes" in pltpu.CompilerParams.__dataclass_fields__:
        cp = dataclasses.replace(cp, needs_layout_passes=False)

    @functools.partial(
        pl.kernel, mesh=mesh, compiler_params=cp,
        out_type=jax.ShapeDtypeStruct((rows, DIM), jnp.float32),
        scratch_types=[
            pltpu.VMEM((rpw, _K), jnp.int32),
            pltpu.VMEM((rpw, _K), jnp.float32),
            pltpu.VMEM((2, _KCH, DIM), jnp.float32),
            pltpu.VMEM((DIM,), jnp.float32),
            pltpu.SemaphoreType.DMA,
            pltpu.SemaphoreType.DMA,
        ],
    )
    def k(values_hbm, vidx_hbm, attn_hbm, out_hbm, idx_v, w_v, gbuf, acc,
          gsem0, gsem1):
        wid = lax.axis_index("s") * _NC + lax.axis_index("c")
        base = wid * rpw
        pltpu.sync_copy(vidx_hbm.at[pl.ds(base, rpw)], idx_v)
        pltpu.sync_copy(attn_hbm.at[pl.ds(base, rpw)], w_v)
        gsems = (gsem0, gsem1)

        def start_gather(r, kb_static_slot, idx_slice):
            pltpu.async_copy(values_hbm.at[idx_slice],
                             gbuf.at[kb_static_slot],
                             gsems[kb_static_slot])

        @pl.loop(0, rpw)
        def _(r):
            @pl.loop(0, DIM // 16)
            def _(c):
                acc[pl.ds(c * 16, 16)] = jnp.zeros((16,), jnp.float32)

            start_gather(r, 0, idx_v.at[r, pl.ds(0, _KCH)])
            for kb in range(_NCHUNK):
                slot = kb % 2
                nslot = (kb + 1) % 2
                if kb + 1 < _NCHUNK:
                    start_gather(r, nslot,
                                 idx_v.at[r, pl.ds((kb + 1) * _KCH, _KCH)])
                pltpu.make_async_copy(values_hbm.at[idx_v.at[r, pl.ds(kb * _KCH, _KCH)]],
                                      gbuf.at[slot], gsems[slot]).wait()
                lane_r = jnp.zeros((16,), jnp.int32) + r
                for k8 in range(_KCH // 8):
                    ws = [plsc.load_gather(
                              w_v, [lane_r,
                                    jnp.full((16,), kb * _KCH + k8 * 8 + j,
                                             jnp.int32)])
                          for j in range(8)]

                    @pl.loop(0, DIM // 16)
                    def _(c, ws=ws, slot=slot, k8=k8):
                        av = acc[pl.ds(c * 16, 16)]
                        for j in range(8):
                            av = av + ws[j] * gbuf[slot, k8 * 8 + j,
                                                   pl.ds(c * 16, 16)]
                        acc[pl.ds(c * 16, 16)] = av

            pltpu.sync_copy(acc, out_hbm.at[base + r])

    return k(values, vidx, attn)


def kernel(x, W_q, gamma, beta, keys, values):
    b, t, e = x.shape
    xs = x.reshape(T, DIM).astype(jnp.bfloat16)
    wqt = W_q.T.astype(jnp.bfloat16)
    keys_t = jnp.transpose(keys, (0, 2, 3, 1)).astype(jnp.bfloat16)
    gamma2d = gamma.reshape(1, DIM_QUERY)
    beta2d = beta.reshape(1, DIM_QUERY)

    q, stats = _compute_q_stats(xs, wqt)
    half = T // 2
    outs = []
    for i in range(2):
        qh = lax.slice(q, (i * half, 0), ((i + 1) * half, DIM_QUERY))
        vidx, attn = _compute_routing(qh, stats, gamma2d, beta2d, keys_t,
                                      rows=half)
        outs.append(_ebag_sc(values, vidx, attn, rows=half))
    out = jnp.concatenate(outs, axis=0)
    return out.reshape(b, t, e)


# speedup vs baseline: 5.0835x; 1.0455x over previous
import dataclasses
import functools
import math

import jax
import jax.numpy as jnp
from jax import lax
from jax.experimental import pallas as pl
from jax.experimental.pallas import tpu as pltpu
from jax.experimental.pallas import tpu_sc as plsc

DIM = 1024
HEADS = 4
NUM_KEYS = 256
TOPK = 32
HALF = 128
DIM_QUERY = 1024
T = 2048
TB = 256
NCAND = 128

_CAND_COUNTS = [min(TOPK, TOPK // (i + 1)) for i in range(TOPK)]


def _qstats_body(x_ref, wqt_ref, q_ref, stats_ref):
    i = pl.program_id(0)
    q = jax.lax.dot_general(
        x_ref[...], wqt_ref[...], (((1,), (0,)), ((), ())),
        preferred_element_type=jnp.float32)
    q_ref[...] = q
    s = jnp.sum(q, axis=0, keepdims=True)
    s2 = jnp.sum(q * q, axis=0, keepdims=True)

    @pl.when(i == 0)
    def _():
        stats_ref[0:1, :] = s
        stats_ref[1:2, :] = s2

    @pl.when(i > 0)
    def _():
        stats_ref[0:1, :] += s
        stats_ref[1:2, :] += s2


def _compute_q_stats(xs, wqt):
    return pl.pallas_call(
        _qstats_body,
        grid=(T // TB,),
        in_specs=[
            pl.BlockSpec((TB, DIM), lambda i: (i, 0)),
            pl.BlockSpec((DIM, DIM_QUERY), lambda i: (0, 0)),
        ],
        out_specs=[
            pl.BlockSpec((TB, DIM_QUERY), lambda i: (i, 0)),
            pl.BlockSpec((2, DIM_QUERY), lambda i: (0, 0)),
        ],
        out_shape=[
            jax.ShapeDtypeStruct((T, DIM_QUERY), jnp.float32),
            jax.ShapeDtypeStruct((2, DIM_QUERY), jnp.float32),
        ],
    )(xs, wqt)


def _topk_extract_ref(A_ref, k, vals_ref, cols_ref=None,
                      V_ref=None, vsel_ref=None):
    R, W = A_ref.shape
    iota = lax.broadcasted_iota(jnp.int32, (R, W), 1)
    kiota = lax.broadcasted_iota(jnp.int32, (R, k), 1)

    def body(i, _):
        A = A_ref[...]
        m = jnp.max(A, axis=1, keepdims=True)
        c = jnp.min(jnp.where(A == m, iota, W), axis=1, keepdims=True)
        vals_ref[...] = jnp.where(kiota == i, m, vals_ref[...])
        if cols_ref is not None:
            cols_ref[...] = jnp.where(kiota == i, c, cols_ref[...])
        if V_ref is not None:
            vv = jnp.sum(jnp.where(iota == c, V_ref[...], 0.0),
                         axis=1, keepdims=True)
            vsel_ref[...] = jnp.where(kiota == i, vv, vsel_ref[...])
        A_ref[...] = jnp.where(iota == c, -jnp.inf, A)
        return 0

    lax.fori_loop(0, k, body, 0)


def _scores_body(q_ref, stats_ref, gamma_ref, beta_ref, keys_ref, ij_ref,
                 vidx_ref, attn_ref,
                 A1_ref, sv_ref, sc_ref, S4_ref, V4_ref, fv_ref, vs_ref):
    mean = stats_ref[0:1, :] / T
    var = stats_ref[1:2, :] / T - mean * mean
    scale = gamma_ref[...] * lax.rsqrt(var + 1e-5)
    qn = (q_ref[...] - mean) * scale + beta_ref[...]

    for h in range(HEADS):
        for p in range(2):
            qs = qn[:, p * 512 + h * HALF: p * 512 + h * HALF + HALF]
            kt = keys_ref[h, p, :, :]
            g = 2 * h + p
            A1_ref[g * TB:(g + 1) * TB, :] = jax.lax.dot_general(
                qs.astype(jnp.bfloat16), kt, (((1,), (0,)), ((), ())),
                preferred_element_type=jnp.float32)
    _topk_extract_ref(A1_ref, TOPK, sv_ref, cols_ref=sc_ref)

    svals = sv_ref[...]
    scols = sc_ref[...]
    Iarr = ij_ref[0, :]
    Jarr = ij_ref[1, :]
    ninf = jnp.full((4 * TB, 1), -jnp.inf, jnp.float32)
    for h in range(HEADS):
        s1 = svals[(2 * h) * TB:(2 * h + 1) * TB, :]
        s2 = svals[(2 * h + 1) * TB:(2 * h + 2) * TB, :]
        i1f = scols[(2 * h) * TB:(2 * h + 1) * TB, :].astype(jnp.float32)
        i2f = scols[(2 * h + 1) * TB:(2 * h + 2) * TB, :].astype(jnp.float32)
        s1p = jnp.concatenate([s1, ninf[:TB]], axis=1)
        i1p = jnp.concatenate([i1f, jnp.zeros((TB, 1), jnp.float32)], axis=1)
        Ib = jnp.broadcast_to(Iarr[None, :], (TB, NCAND))
        Jb = jnp.broadcast_to(Jarr[None, :], (TB, NCAND))
        S4_ref[h * TB:(h + 1) * TB, :] = (
            jnp.take_along_axis(s1p, Ib, axis=1)
            + jnp.take_along_axis(s2, Jb, axis=1))
        V4_ref[h * TB:(h + 1) * TB, :] = (
            jnp.take_along_axis(i1p, Ib, axis=1) * NUM_KEYS
            + jnp.take_along_axis(i2f, Jb, axis=1))

    _topk_extract_ref(S4_ref, TOPK, fv_ref, V_ref=V4_ref, vsel_ref=vs_ref)

    fvals = fv_ref[...]
    e = jnp.exp(fvals - fvals[:, 0:1])
    attn4 = e / jnp.sum(e, axis=1, keepdims=True)
    vsel4 = jnp.clip(vs_ref[...], 0.0, float(NUM_KEYS * NUM_KEYS - 1))
    attn_ref[...] = jnp.concatenate(
        [attn4[h * TB:(h + 1) * TB, :] for h in range(HEADS)], axis=1)
    vidx_ref[...] = jnp.concatenate(
        [vsel4[h * TB:(h + 1) * TB, :] for h in range(HEADS)],
        axis=1).astype(jnp.int32)


def _cand_ij():
    Ilist = []
    Jlist = []
    for i, ni in enumerate(_CAND_COUNTS):
        Ilist += [i] * ni
        Jlist += list(range(ni))
    padn = NCAND - len(Ilist)
    Ilist += [TOPK] * padn
    Jlist += [0] * padn
    import numpy as _np
    return jnp.asarray(_np.stack([_np.asarray(Ilist, _np.int32),
                                  _np.asarray(Jlist, _np.int32)]))


def _compute_routing(q, stats, gamma2d, beta2d, keys_t, rows=T):
    nblk = rows // TB
    ij = _cand_ij()
    return pl.pallas_call(
        _scores_body,
        grid=(nblk,),
        in_specs=[
            pl.BlockSpec((TB, DIM_QUERY), lambda i: (i, 0)),
            pl.BlockSpec((2, DIM_QUERY), lambda i: (0, 0)),
            pl.BlockSpec((1, DIM_QUERY), lambda i: (0, 0)),
            pl.BlockSpec((1, DIM_QUERY), lambda i: (0, 0)),
            pl.BlockSpec((HEADS, 2, HALF, NUM_KEYS), lambda i: (0, 0, 0, 0)),
            pl.BlockSpec((2, NCAND), lambda i: (0, 0)),
        ],
        out_specs=[
            pl.BlockSpec((TB, HEADS * TOPK), lambda i: (i, 0)),
            pl.BlockSpec((TB, HEADS * TOPK), lambda i: (i, 0)),
        ],
        out_shape=[
            jax.ShapeDtypeStruct((rows, HEADS * TOPK), jnp.int32),
            jax.ShapeDtypeStruct((rows, HEADS * TOPK), jnp.float32),
        ],
        scratch_shapes=[
            pltpu.VMEM((8 * TB, NUM_KEYS), jnp.float32),
            pltpu.VMEM((8 * TB, TOPK), jnp.float32),
            pltpu.VMEM((8 * TB, TOPK), jnp.int32),
            pltpu.VMEM((HEADS * TB, NCAND), jnp.float32),
            pltpu.VMEM((HEADS * TB, NCAND), jnp.float32),
            pltpu.VMEM((HEADS * TB, TOPK), jnp.float32),
            pltpu.VMEM((HEADS * TB, TOPK), jnp.float32),
        ],
    )(q, stats, gamma2d, beta2d, keys_t, ij)


_NC = 2
_NS = 16
_NW = _NC * _NS
_RPW = T // _NW
_K = HEADS * TOPK
_KCH = 32
_NCHUNK = _K // _KCH


def _ebag_sc(values, vidx, attn, rows=T):
    rpw = rows // _NW
    mesh = plsc.VectorSubcoreMesh(core_axis_name="c", subcore_axis_name="s")
    cp = pltpu.CompilerParams()
    if "needs_layout_passes" in pltpu.CompilerParams.__dataclass_fields__:
        cp = dataclasses.replace(cp, needs_layout_passes=False)

    @functools.partial(
        pl.kernel, mesh=mesh, compiler_params=cp,
        out_type=jax.ShapeDtypeStruct((rows, DIM), jnp.float32),
        scratch_types=[
            pltpu.VMEM((rpw, _K), jnp.int32),
            pltpu.VMEM((rpw, _K), jnp.float32),
            pltpu.VMEM((2, _KCH, DIM), jnp.float32),
            pltpu.VMEM((DIM,), jnp.float32),
            pltpu.SemaphoreType.DMA,
            pltpu.SemaphoreType.DMA,
        ],
    )
    def k(values_hbm, vidx_hbm, attn_hbm, out_hbm, idx_v, w_v, gbuf, acc,
          gsem0, gsem1):
        wid = lax.axis_index("s") * _NC + lax.axis_index("c")
        base = wid * rpw
        pltpu.sync_copy(vidx_hbm.at[pl.ds(base, rpw)], idx_v)
        pltpu.sync_copy(attn_hbm.at[pl.ds(base, rpw)], w_v)
        gsems = (gsem0, gsem1)

        def start_gather(r, kb_static_slot, idx_slice):
            pltpu.async_copy(values_hbm.at[idx_slice],
                             gbuf.at[kb_static_slot],
                             gsems[kb_static_slot])

        @pl.loop(0, rpw)
        def _(r):
            @pl.loop(0, DIM // 16)
            def _(c):
                acc[pl.ds(c * 16, 16)] = jnp.zeros((16,), jnp.float32)

            start_gather(r, 0, idx_v.at[r, pl.ds(0, _KCH)])
            for kb in range(_NCHUNK):
                slot = kb % 2
                nslot = (kb + 1) % 2
                if kb + 1 < _NCHUNK:
                    start_gather(r, nslot,
                                 idx_v.at[r, pl.ds((kb + 1) * _KCH, _KCH)])
                pltpu.make_async_copy(values_hbm.at[idx_v.at[r, pl.ds(kb * _KCH, _KCH)]],
                                      gbuf.at[slot], gsems[slot]).wait()
                lane_r = jnp.zeros((16,), jnp.int32) + r
                for k8 in range(_KCH // 8):
                    ws = [plsc.load_gather(
                              w_v, [lane_r,
                                    jnp.full((16,), kb * _KCH + k8 * 8 + j,
                                             jnp.int32)])
                          for j in range(8)]

                    @pl.loop(0, DIM // 16)
                    def _(c, ws=ws, slot=slot, k8=k8):
                        av = acc[pl.ds(c * 16, 16)]
                        for j in range(8):
                            av = av + ws[j] * gbuf[slot, k8 * 8 + j,
                                                   pl.ds(c * 16, 16)]
                        acc[pl.ds(c * 16, 16)] = av

            pltpu.sync_copy(acc, out_hbm.at[base + r])

    return k(values, vidx, attn)


def kernel(x, W_q, gamma, beta, keys, values):
    b, t, e = x.shape
    xs = x.reshape(T, DIM).astype(jnp.bfloat16)
    wqt = W_q.T.astype(jnp.bfloat16)
    keys_t = jnp.transpose(keys, (0, 2, 3, 1)).astype(jnp.bfloat16)
    gamma2d = gamma.reshape(1, DIM_QUERY)
    beta2d = beta.reshape(1, DIM_QUERY)

    q, stats = _compute_q_stats(xs, wqt)
    half = T // 2
    outs = []
    for i in range(2):
        qh = lax.slice(q, (i * half, 0), ((i + 1) * half, DIM_QUERY))
        vidx, attn = _compute_routing(qh, stats, gamma2d, beta2d, keys_t,
                                      rows=half)
        outs.append(_ebag_sc(values, vidx, attn, rows=half))
    out = jnp.concatenate(outs, axis=0)
    return out.reshape(b, t, e)


# speedup vs baseline: 5.3293x; 1.0484x over previous
import dataclasses
import functools
import math

import jax
import jax.numpy as jnp
from jax import lax
from jax.experimental import pallas as pl
from jax.experimental.pallas import tpu as pltpu
from jax.experimental.pallas import tpu_sc as plsc

DIM = 1024
HEADS = 4
NUM_KEYS = 256
TOPK = 32
HALF = 128
DIM_QUERY = 1024
T = 2048
TB = 256
NCAND = 128

_CAND_COUNTS = [min(TOPK, TOPK // (i + 1)) for i in range(TOPK)]


def _qstats_body(x_ref, wqt_ref, q_ref, stats_ref):
    i = pl.program_id(0)
    q = jax.lax.dot_general(
        x_ref[...], wqt_ref[...], (((1,), (0,)), ((), ())),
        preferred_element_type=jnp.float32)
    q_ref[...] = q
    s = jnp.sum(q, axis=0, keepdims=True)
    s2 = jnp.sum(q * q, axis=0, keepdims=True)

    @pl.when(i == 0)
    def _():
        stats_ref[0:1, :] = s
        stats_ref[1:2, :] = s2

    @pl.when(i > 0)
    def _():
        stats_ref[0:1, :] += s
        stats_ref[1:2, :] += s2


def _compute_q_stats(xs, wqt):
    return pl.pallas_call(
        _qstats_body,
        grid=(T // TB,),
        in_specs=[
            pl.BlockSpec((TB, DIM), lambda i: (i, 0)),
            pl.BlockSpec((DIM, DIM_QUERY), lambda i: (0, 0)),
        ],
        out_specs=[
            pl.BlockSpec((TB, DIM_QUERY), lambda i: (i, 0)),
            pl.BlockSpec((2, DIM_QUERY), lambda i: (0, 0)),
        ],
        out_shape=[
            jax.ShapeDtypeStruct((T, DIM_QUERY), jnp.float32),
            jax.ShapeDtypeStruct((2, DIM_QUERY), jnp.float32),
        ],
    )(xs, wqt)


def _topk_extract_ref(A_ref, k, vals_ref, cols_ref=None,
                      V_ref=None, vsel_ref=None):
    R, W = A_ref.shape
    iota = lax.broadcasted_iota(jnp.int32, (R, W), 1)
    kiota = lax.broadcasted_iota(jnp.int32, (R, k), 1)

    def body(i, _):
        A = A_ref[...]
        m = jnp.max(A, axis=1, keepdims=True)
        c = jnp.min(jnp.where(A == m, iota, W), axis=1, keepdims=True)
        vals_ref[...] = jnp.where(kiota == i, m, vals_ref[...])
        if cols_ref is not None:
            cols_ref[...] = jnp.where(kiota == i, c, cols_ref[...])
        if V_ref is not None:
            vv = jnp.sum(jnp.where(iota == c, V_ref[...], 0.0),
                         axis=1, keepdims=True)
            vsel_ref[...] = jnp.where(kiota == i, vv, vsel_ref[...])
        A_ref[...] = jnp.where(iota == c, -jnp.inf, A)
        return 0

    lax.fori_loop(0, k, body, 0)


def _scores_body(q_ref, stats_ref, gamma_ref, beta_ref, keys_ref, ij_ref,
                 vidx_ref, attn_ref,
                 A1_ref, sv_ref, sc_ref, S4_ref, V4_ref, fv_ref, vs_ref):
    mean = stats_ref[0:1, :] / T
    var = stats_ref[1:2, :] / T - mean * mean
    scale = gamma_ref[...] * lax.rsqrt(var + 1e-5)
    qn = (q_ref[...] - mean) * scale + beta_ref[...]

    for h in range(HEADS):
        for p in range(2):
            qs = qn[:, p * 512 + h * HALF: p * 512 + h * HALF + HALF]
            kt = keys_ref[h, p, :, :]
            g = 2 * h + p
            A1_ref[g * TB:(g + 1) * TB, :] = jax.lax.dot_general(
                qs.astype(jnp.bfloat16), kt, (((1,), (0,)), ((), ())),
                preferred_element_type=jnp.float32)
    _topk_extract_ref(A1_ref, TOPK, sv_ref, cols_ref=sc_ref)

    svals = sv_ref[...]
    scols = sc_ref[...]
    Iarr = ij_ref[0, :]
    Jarr = ij_ref[1, :]
    ninf = jnp.full((4 * TB, 1), -jnp.inf, jnp.float32)
    for h in range(HEADS):
        s1 = svals[(2 * h) * TB:(2 * h + 1) * TB, :]
        s2 = svals[(2 * h + 1) * TB:(2 * h + 2) * TB, :]
        i1f = scols[(2 * h) * TB:(2 * h + 1) * TB, :].astype(jnp.float32)
        i2f = scols[(2 * h + 1) * TB:(2 * h + 2) * TB, :].astype(jnp.float32)
        s1p = jnp.concatenate([s1, ninf[:TB]], axis=1)
        i1p = jnp.concatenate([i1f, jnp.zeros((TB, 1), jnp.float32)], axis=1)
        Ib = jnp.broadcast_to(Iarr[None, :], (TB, NCAND))
        Jb = jnp.broadcast_to(Jarr[None, :], (TB, NCAND))
        S4_ref[h * TB:(h + 1) * TB, :] = (
            jnp.take_along_axis(s1p, Ib, axis=1)
            + jnp.take_along_axis(s2, Jb, axis=1))
        V4_ref[h * TB:(h + 1) * TB, :] = (
            jnp.take_along_axis(i1p, Ib, axis=1) * NUM_KEYS
            + jnp.take_along_axis(i2f, Jb, axis=1))

    _topk_extract_ref(S4_ref, TOPK, fv_ref, V_ref=V4_ref, vsel_ref=vs_ref)

    fvals = fv_ref[...]
    e = jnp.exp(fvals - fvals[:, 0:1])
    attn4 = e / jnp.sum(e, axis=1, keepdims=True)
    vsel4 = jnp.clip(vs_ref[...], 0.0, float(NUM_KEYS * NUM_KEYS - 1))
    attn_ref[...] = jnp.concatenate(
        [attn4[h * TB:(h + 1) * TB, :] for h in range(HEADS)], axis=1)
    vidx_ref[...] = jnp.concatenate(
        [vsel4[h * TB:(h + 1) * TB, :] for h in range(HEADS)],
        axis=1).astype(jnp.int32)


def _cand_ij():
    Ilist = []
    Jlist = []
    for i, ni in enumerate(_CAND_COUNTS):
        Ilist += [i] * ni
        Jlist += list(range(ni))
    padn = NCAND - len(Ilist)
    Ilist += [TOPK] * padn
    Jlist += [0] * padn
    import numpy as _np
    return jnp.asarray(_np.stack([_np.asarray(Ilist, _np.int32),
                                  _np.asarray(Jlist, _np.int32)]))


def _compute_routing(q, stats, gamma2d, beta2d, keys_t, rows=T):
    nblk = rows // TB
    ij = _cand_ij()
    return pl.pallas_call(
        _scores_body,
        grid=(nblk,),
        in_specs=[
            pl.BlockSpec((TB, DIM_QUERY), lambda i: (i, 0)),
            pl.BlockSpec((2, DIM_QUERY), lambda i: (0, 0)),
            pl.BlockSpec((1, DIM_QUERY), lambda i: (0, 0)),
            pl.BlockSpec((1, DIM_QUERY), lambda i: (0, 0)),
            pl.BlockSpec((HEADS, 2, HALF, NUM_KEYS), lambda i: (0, 0, 0, 0)),
            pl.BlockSpec((2, NCAND), lambda i: (0, 0)),
        ],
        out_specs=[
            pl.BlockSpec((TB, HEADS * TOPK), lambda i: (i, 0)),
            pl.BlockSpec((TB, HEADS * TOPK), lambda i: (i, 0)),
        ],
        out_shape=[
            jax.ShapeDtypeStruct((rows, HEADS * TOPK), jnp.int32),
            jax.ShapeDtypeStruct((rows, HEADS * TOPK), jnp.float32),
        ],
        scratch_shapes=[
            pltpu.VMEM((8 * TB, NUM_KEYS), jnp.float32),
            pltpu.VMEM((8 * TB, TOPK), jnp.float32),
            pltpu.VMEM((8 * TB, TOPK), jnp.int32),
            pltpu.VMEM((HEADS * TB, NCAND), jnp.float32),
            pltpu.VMEM((HEADS * TB, NCAND), jnp.float32),
            pltpu.VMEM((HEADS * TB, TOPK), jnp.float32),
            pltpu.VMEM((HEADS * TB, TOPK), jnp.float32),
        ],
    )(q, stats, gamma2d, beta2d, keys_t, ij)


_NC = 2
_NS = 16
_NW = _NC * _NS
_RPW = T // _NW
_K = HEADS * TOPK
_KCH = 32
_NCHUNK = _K // _KCH


def _ebag_sc(values, vidx, attn, rows=T):
    rpw = rows // _NW
    mesh = plsc.VectorSubcoreMesh(core_axis_name="c", subcore_axis_name="s")
    cp = pltpu.CompilerParams()
    if "needs_layout_passes" in pltpu.CompilerParams.__dataclass_fields__:
        cp = dataclasses.replace(cp, needs_layout_passes=False)

    @functools.partial(
        pl.kernel, mesh=mesh, compiler_params=cp,
        out_type=jax.ShapeDtypeStruct((rows, DIM), jnp.float32),
        scratch_types=[
            pltpu.VMEM((rpw, _K), jnp.int32),
            pltpu.VMEM((rpw, _K), jnp.float32),
            pltpu.VMEM((2, _KCH, DIM), jnp.float32),
            pltpu.VMEM((DIM,), jnp.float32),
            pltpu.SemaphoreType.DMA,
            pltpu.SemaphoreType.DMA,
        ],
    )
    def k(values_hbm, vidx_hbm, attn_hbm, out_hbm, idx_v, w_v, gbuf, acc,
          gsem0, gsem1):
        wid = lax.axis_index("s") * _NC + lax.axis_index("c")
        base = wid * rpw
        pltpu.sync_copy(vidx_hbm.at[pl.ds(base, rpw)], idx_v)
        pltpu.sync_copy(attn_hbm.at[pl.ds(base, rpw)], w_v)
        gsems = (gsem0, gsem1)

        def start_gather(r, kb_static_slot, idx_slice):
            pltpu.async_copy(values_hbm.at[idx_slice],
                             gbuf.at[kb_static_slot],
                             gsems[kb_static_slot])

        @pl.loop(0, rpw)
        def _(r):
            @pl.loop(0, DIM // 16)
            def _(c):
                acc[pl.ds(c * 16, 16)] = jnp.zeros((16,), jnp.float32)

            start_gather(r, 0, idx_v.at[r, pl.ds(0, _KCH)])
            for kb in range(_NCHUNK):
                slot = kb % 2
                nslot = (kb + 1) % 2
                if kb + 1 < _NCHUNK:
                    start_gather(r, nslot,
                                 idx_v.at[r, pl.ds((kb + 1) * _KCH, _KCH)])
                pltpu.make_async_copy(values_hbm.at[idx_v.at[r, pl.ds(kb * _KCH, _KCH)]],
                                      gbuf.at[slot], gsems[slot]).wait()
                lane_r = jnp.zeros((16,), jnp.int32) + r
                for k8 in range(_KCH // 8):
                    ws = [plsc.load_gather(
                              w_v, [lane_r,
                                    jnp.full((16,), kb * _KCH + k8 * 8 + j,
                                             jnp.int32)])
                          for j in range(8)]

                    @pl.loop(0, DIM // 16)
                    def _(c, ws=ws, slot=slot, k8=k8):
                        sl = pl.ds(c * 16, 16)
                        g = [gbuf[slot, k8 * 8 + j, sl] for j in range(8)]
                        p0 = ws[0] * g[0] + ws[4] * g[4]
                        p1 = ws[1] * g[1] + ws[5] * g[5]
                        p2 = ws[2] * g[2] + ws[6] * g[6]
                        p3 = ws[3] * g[3] + ws[7] * g[7]
                        acc[sl] = acc[sl] + ((p0 + p1) + (p2 + p3))

            pltpu.sync_copy(acc, out_hbm.at[base + r])

    return k(values, vidx, attn)


def kernel(x, W_q, gamma, beta, keys, values):
    b, t, e = x.shape
    xs = x.reshape(T, DIM).astype(jnp.bfloat16)
    wqt = W_q.T.astype(jnp.bfloat16)
    keys_t = jnp.transpose(keys, (0, 2, 3, 1)).astype(jnp.bfloat16)
    gamma2d = gamma.reshape(1, DIM_QUERY)
    beta2d = beta.reshape(1, DIM_QUERY)

    q, stats = _compute_q_stats(xs, wqt)
    half = T // 2
    outs = []
    for i in range(2):
        qh = lax.slice(q, (i * half, 0), ((i + 1) * half, DIM_QUERY))
        vidx, attn = _compute_routing(qh, stats, gamma2d, beta2d, keys_t,
                                      rows=half)
        outs.append(_ebag_sc(values, vidx, attn, rows=half))
    out = jnp.concatenate(outs, axis=0)
    return out.reshape(b, t, e)


# speedup vs baseline: 5.7207x; 1.0735x over previous
import dataclasses
import functools
import math

import jax
import jax.numpy as jnp
from jax import lax
from jax.experimental import pallas as pl
from jax.experimental.pallas import tpu as pltpu
from jax.experimental.pallas import tpu_sc as plsc

DIM = 1024
HEADS = 4
NUM_KEYS = 256
TOPK = 32
HALF = 128
DIM_QUERY = 1024
T = 2048
TB = 256
NCAND = 128

_CAND_COUNTS = [min(TOPK, TOPK // (i + 1)) for i in range(TOPK)]


def _qstats_body(x_ref, wqt_ref, q_ref, stats_ref):
    i = pl.program_id(0)
    q = jax.lax.dot_general(
        x_ref[...], wqt_ref[...], (((1,), (0,)), ((), ())),
        preferred_element_type=jnp.float32)
    q_ref[...] = q
    s = jnp.sum(q, axis=0, keepdims=True)
    s2 = jnp.sum(q * q, axis=0, keepdims=True)

    @pl.when(i == 0)
    def _():
        stats_ref[0:1, :] = s
        stats_ref[1:2, :] = s2

    @pl.when(i > 0)
    def _():
        stats_ref[0:1, :] += s
        stats_ref[1:2, :] += s2


def _compute_q_stats(xs, wqt):
    return pl.pallas_call(
        _qstats_body,
        grid=(T // TB,),
        in_specs=[
            pl.BlockSpec((TB, DIM), lambda i: (i, 0)),
            pl.BlockSpec((DIM, DIM_QUERY), lambda i: (0, 0)),
        ],
        out_specs=[
            pl.BlockSpec((TB, DIM_QUERY), lambda i: (i, 0)),
            pl.BlockSpec((2, DIM_QUERY), lambda i: (0, 0)),
        ],
        out_shape=[
            jax.ShapeDtypeStruct((T, DIM_QUERY), jnp.float32),
            jax.ShapeDtypeStruct((2, DIM_QUERY), jnp.float32),
        ],
    )(xs, wqt)


def _topk_extract_ref(A_ref, k, vals_ref, cols_ref=None,
                      V_ref=None, vsel_ref=None):
    R, W = A_ref.shape
    iota = lax.broadcasted_iota(jnp.int32, (R, W), 1)
    kiota = lax.broadcasted_iota(jnp.int32, (R, k), 1)

    def body(i, _):
        A = A_ref[...]
        m = jnp.max(A, axis=1, keepdims=True)
        c = jnp.min(jnp.where(A == m, iota, W), axis=1, keepdims=True)
        vals_ref[...] = jnp.where(kiota == i, m, vals_ref[...])
        if cols_ref is not None:
            cols_ref[...] = jnp.where(kiota == i, c, cols_ref[...])
        if V_ref is not None:
            vv = jnp.sum(jnp.where(iota == c, V_ref[...], 0.0),
                         axis=1, keepdims=True)
            vsel_ref[...] = jnp.where(kiota == i, vv, vsel_ref[...])
        A_ref[...] = jnp.where(iota == c, -jnp.inf, A)
        return 0

    lax.fori_loop(0, k, body, 0)


def _scores_body(q_ref, stats_ref, gamma_ref, beta_ref, keys_ref, ij_ref,
                 vidx_ref, attn_ref,
                 A1_ref, sv_ref, sc_ref, S4_ref, V4_ref, fv_ref, vs_ref):
    mean = stats_ref[0:1, :] / T
    var = stats_ref[1:2, :] / T - mean * mean
    scale = gamma_ref[...] * lax.rsqrt(var + 1e-5)
    qn = (q_ref[...] - mean) * scale + beta_ref[...]

    for h in range(HEADS):
        for p in range(2):
            qs = qn[:, p * 512 + h * HALF: p * 512 + h * HALF + HALF]
            kt = keys_ref[h, p, :, :]
            g = 2 * h + p
            A1_ref[g * TB:(g + 1) * TB, :] = jax.lax.dot_general(
                qs.astype(jnp.bfloat16), kt, (((1,), (0,)), ((), ())),
                preferred_element_type=jnp.float32)
    _topk_extract_ref(A1_ref, TOPK, sv_ref, cols_ref=sc_ref)

    svals = sv_ref[...]
    scols = sc_ref[...]
    Iarr = ij_ref[0, :]
    Jarr = ij_ref[1, :]
    ninf = jnp.full((4 * TB, 1), -jnp.inf, jnp.float32)
    for h in range(HEADS):
        s1 = svals[(2 * h) * TB:(2 * h + 1) * TB, :]
        s2 = svals[(2 * h + 1) * TB:(2 * h + 2) * TB, :]
        i1f = scols[(2 * h) * TB:(2 * h + 1) * TB, :].astype(jnp.float32)
        i2f = scols[(2 * h + 1) * TB:(2 * h + 2) * TB, :].astype(jnp.float32)
        s1p = jnp.concatenate([s1, ninf[:TB]], axis=1)
        i1p = jnp.concatenate([i1f, jnp.zeros((TB, 1), jnp.float32)], axis=1)
        Ib = jnp.broadcast_to(Iarr[None, :], (TB, NCAND))
        Jb = jnp.broadcast_to(Jarr[None, :], (TB, NCAND))
        S4_ref[h * TB:(h + 1) * TB, :] = (
            jnp.take_along_axis(s1p, Ib, axis=1)
            + jnp.take_along_axis(s2, Jb, axis=1))
        V4_ref[h * TB:(h + 1) * TB, :] = (
            jnp.take_along_axis(i1p, Ib, axis=1) * NUM_KEYS
            + jnp.take_along_axis(i2f, Jb, axis=1))

    _topk_extract_ref(S4_ref, TOPK, fv_ref, V_ref=V4_ref, vsel_ref=vs_ref)

    fvals = fv_ref[...]
    e = jnp.exp(fvals - fvals[:, 0:1])
    attn4 = e / jnp.sum(e, axis=1, keepdims=True)
    vsel4 = jnp.clip(vs_ref[...], 0.0, float(NUM_KEYS * NUM_KEYS - 1))
    attn_ref[...] = jnp.concatenate(
        [attn4[h * TB:(h + 1) * TB, :] for h in range(HEADS)], axis=1)
    vidx_ref[...] = jnp.concatenate(
        [vsel4[h * TB:(h + 1) * TB, :] for h in range(HEADS)],
        axis=1).astype(jnp.int32)


def _cand_ij():
    Ilist = []
    Jlist = []
    for i, ni in enumerate(_CAND_COUNTS):
        Ilist += [i] * ni
        Jlist += list(range(ni))
    padn = NCAND - len(Ilist)
    Ilist += [TOPK] * padn
    Jlist += [0] * padn
    import numpy as _np
    return jnp.asarray(_np.stack([_np.asarray(Ilist, _np.int32),
                                  _np.asarray(Jlist, _np.int32)]))


def _compute_routing(q, stats, gamma2d, beta2d, keys_t, rows=T):
    nblk = rows // TB
    ij = _cand_ij()
    return pl.pallas_call(
        _scores_body,
        grid=(nblk,),
        in_specs=[
            pl.BlockSpec((TB, DIM_QUERY), lambda i: (i, 0)),
            pl.BlockSpec((2, DIM_QUERY), lambda i: (0, 0)),
            pl.BlockSpec((1, DIM_QUERY), lambda i: (0, 0)),
            pl.BlockSpec((1, DIM_QUERY), lambda i: (0, 0)),
            pl.BlockSpec((HEADS, 2, HALF, NUM_KEYS), lambda i: (0, 0, 0, 0)),
            pl.BlockSpec((2, NCAND), lambda i: (0, 0)),
        ],
        out_specs=[
            pl.BlockSpec((TB, HEADS * TOPK), lambda i: (i, 0)),
            pl.BlockSpec((TB, HEADS * TOPK), lambda i: (i, 0)),
        ],
        out_shape=[
            jax.ShapeDtypeStruct((rows, HEADS * TOPK), jnp.int32),
            jax.ShapeDtypeStruct((rows, HEADS * TOPK), jnp.float32),
        ],
        scratch_shapes=[
            pltpu.VMEM((8 * TB, NUM_KEYS), jnp.float32),
            pltpu.VMEM((8 * TB, TOPK), jnp.float32),
            pltpu.VMEM((8 * TB, TOPK), jnp.int32),
            pltpu.VMEM((HEADS * TB, NCAND), jnp.float32),
            pltpu.VMEM((HEADS * TB, NCAND), jnp.float32),
            pltpu.VMEM((HEADS * TB, TOPK), jnp.float32),
            pltpu.VMEM((HEADS * TB, TOPK), jnp.float32),
        ],
    )(q, stats, gamma2d, beta2d, keys_t, ij)


_NC = 2
_NS = 16
_NW = _NC * _NS
_RPW = T // _NW
_K = HEADS * TOPK
_KCH = 32
_NCHUNK = _K // _KCH


def _ebag_sc(values, vidx, attn, rows=T):
    rpw = rows // _NW
    mesh = plsc.VectorSubcoreMesh(core_axis_name="c", subcore_axis_name="s")
    cp = pltpu.CompilerParams()
    if "needs_layout_passes" in pltpu.CompilerParams.__dataclass_fields__:
        cp = dataclasses.replace(cp, needs_layout_passes=False)

    @functools.partial(
        pl.kernel, mesh=mesh, compiler_params=cp,
        out_type=jax.ShapeDtypeStruct((rows, DIM), jnp.float32),
        scratch_types=[
            pltpu.VMEM((rpw, _K), jnp.int32),
            pltpu.VMEM((rpw, _K), jnp.float32),
            pltpu.VMEM((2, _KCH, DIM), jnp.float32),
            pltpu.VMEM((DIM,), jnp.float32),
            pltpu.SemaphoreType.DMA,
            pltpu.SemaphoreType.DMA,
        ],
    )
    def k(values_hbm, vidx_hbm, attn_hbm, out_hbm, idx_v, w_v, gbuf, acc,
          gsem0, gsem1):
        wid = lax.axis_index("s") * _NC + lax.axis_index("c")
        base = wid * rpw
        pltpu.sync_copy(vidx_hbm.at[pl.ds(base, rpw)], idx_v)
        pltpu.sync_copy(attn_hbm.at[pl.ds(base, rpw)], w_v)
        gsems = (gsem0, gsem1)

        def start_gather(r, kb_static_slot, idx_slice):
            pltpu.async_copy(values_hbm.at[idx_slice],
                             gbuf.at[kb_static_slot],
                             gsems[kb_static_slot])

        @pl.loop(0, rpw)
        def _(r):
            @pl.loop(0, DIM // 16)
            def _(c):
                acc[pl.ds(c * 16, 16)] = jnp.zeros((16,), jnp.float32)

            start_gather(r, 0, idx_v.at[r, pl.ds(0, _KCH)])
            for kb in range(_NCHUNK):
                slot = kb % 2
                nslot = (kb + 1) % 2
                if kb + 1 < _NCHUNK:
                    start_gather(r, nslot,
                                 idx_v.at[r, pl.ds((kb + 1) * _KCH, _KCH)])
                pltpu.make_async_copy(values_hbm.at[idx_v.at[r, pl.ds(kb * _KCH, _KCH)]],
                                      gbuf.at[slot], gsems[slot]).wait()
                lane_r = jnp.zeros((16,), jnp.int32) + r
                for k8 in range(_KCH // 8):
                    ws = [plsc.load_gather(
                              w_v, [lane_r,
                                    jnp.full((16,), kb * _KCH + k8 * 8 + j,
                                             jnp.int32)])
                          for j in range(8)]

                    @pl.loop(0, DIM // 16)
                    def _(c, ws=ws, slot=slot, k8=k8):
                        sl = pl.ds(c * 16, 16)
                        g = [gbuf[slot, k8 * 8 + j, sl] for j in range(8)]
                        p0 = ws[0] * g[0] + ws[4] * g[4]
                        p1 = ws[1] * g[1] + ws[5] * g[5]
                        p2 = ws[2] * g[2] + ws[6] * g[6]
                        p3 = ws[3] * g[3] + ws[7] * g[7]
                        acc[sl] = acc[sl] + ((p0 + p1) + (p2 + p3))

            pltpu.sync_copy(acc, out_hbm.at[base + r])

    return k(values, vidx, attn)


def kernel(x, W_q, gamma, beta, keys, values):
    b, t, e = x.shape
    xs = x.reshape(T, DIM).astype(jnp.bfloat16)
    wqt = W_q.T.astype(jnp.bfloat16)
    keys_t = jnp.transpose(keys, (0, 2, 3, 1)).astype(jnp.bfloat16)
    gamma2d = gamma.reshape(1, DIM_QUERY)
    beta2d = beta.reshape(1, DIM_QUERY)

    q, stats = _compute_q_stats(xs, wqt)
    part = T // 4
    outs = []
    for i in range(4):
        qh = lax.slice(q, (i * part, 0), ((i + 1) * part, DIM_QUERY))
        vidx, attn = _compute_routing(qh, stats, gamma2d, beta2d, keys_t,
                                      rows=part)
        outs.append(_ebag_sc(values, vidx, attn, rows=part))
    out = jnp.concatenate(outs, axis=0)
    return out.reshape(b, t, e)
